# baseline (speedup 1.0000x reference)
import functools

import jax
import jax.numpy as jnp
from jax import lax
from jax.experimental import pallas as pl
from jax.experimental.pallas import tpu as pltpu
from jax.experimental.pallas import tpu_sc as plsc

N = 16384
D = 32
DP = 33
C = 1000
CP = 1024
NS = 16
R = N // NS
CHUNK = 128
NCHUNK = R // CHUNK
L = 16


def _lanes_f32(val):
    return jnp.full((L,), val, dtype=jnp.float32)


def _sc_body(emb_hbm, tgt_hbm, out_hbm,
             emb_v, emb33_v, tgt_v, sums_l, cnts_l, ones_v, zrow_v, zcnt_v,
             part_v, sums_sh, cnts_sh, sem_e, sem_p, sem_t, sem_s):
    wid = lax.axis_index("s")
    base = wid * R
    zero16 = _lanes_f32(0.0)

    cp_emb = pltpu.async_copy(emb_hbm.at[pl.ds(base, R), :], emb_v, sem_e)
    cp_pad = pltpu.async_copy(emb_hbm.at[pl.ds(base, R), :],
                              emb33_v.at[:, pl.ds(0, D)], sem_p)
    cp_tgt = pltpu.async_copy(tgt_hbm.at[wid], tgt_v, sem_t)

    for i in range(CP // NS):
        zrow_v[i, pl.ds(0, L)] = zero16
        zrow_v[i, pl.ds(L, L)] = zero16
    for i in range((CP // NS) // L):
        zcnt_v[pl.ds(i * L, L)] = zero16
    for i in range(CHUNK // L):
        ones_v[pl.ds(i * L, L)] = _lanes_f32(1.0)
    pltpu.sync_copy(zrow_v, sums_sh.at[pl.ds(wid * (CP // NS), CP // NS), :])
    pltpu.sync_copy(zcnt_v, cnts_sh.at[pl.ds(wid * (CP // NS), CP // NS)])
    cp_emb.wait()
    cp_tgt.wait()
    plsc.subcore_barrier()

    adds = []
    for j in range(NCHUNK):
        idx = tgt_v.at[j]
        adds.append(pltpu.async_copy(emb_v.at[pl.ds(j * CHUNK, CHUNK), :],
                                     sums_sh.at[idx], sem_s, add=True))
        adds.append(pltpu.async_copy(ones_v, cnts_sh.at[idx], sem_s,
                                     add=True))
    for a in adds:
        a.wait()
    plsc.subcore_barrier()

    pltpu.sync_copy(sums_sh, sums_l)
    pltpu.sync_copy(cnts_sh, cnts_l)
    cp_pad.wait()

    lane = lax.broadcasted_iota(jnp.int32, (L,), 0)

    def row_block(rv):
        tv = plsc.load_gather(tgt_v, [lax.shift_right_logical(rv, 7),
                                      rv & (CHUNK - 1)])
        cv = plsc.load_gather(cnts_l, [tv])
        invc = 1.0 / jnp.maximum(cv, 1.0)
        sq = [_lanes_f32(0.0) for _ in range(4)]
        for d in range(D):
            dv = jnp.full((L,), d, dtype=jnp.int32)
            ev = plsc.load_gather(emb33_v, [rv, dv])
            sv = plsc.load_gather(sums_l, [tv, dv])
            diff = ev - invc * sv
            sq[d & 3] = sq[d & 3] + diff * diff
        s = (sq[0] + sq[1]) + (sq[2] + sq[3])
        i = plsc.bitcast(s, jnp.int32)
        i = 0x5F3759DF - lax.shift_right_logical(i, 1)
        y = plsc.bitcast(i, jnp.float32)
        for _ in range(3):
            y = y * (1.5 - 0.5 * s * y * y)
        norm = jnp.where(s > 0.0, s * y, 0.0)
        return norm * invc

    def group_body(g, acc):
        r0 = g * (2 * L) + lane
        return acc + row_block(r0) + row_block(r0 + L)

    part = lax.fori_loop(0, R // (2 * L), group_body, _lanes_f32(0.0))
    part_v[...] = part
    pltpu.sync_copy(part_v, out_hbm.at[pl.ds(wid * L, L)])


@jax.jit
def _center_loss_sc(embeddeds, tgt3d):
    mesh = plsc.VectorSubcoreMesh(core_axis_name="c", subcore_axis_name="s",
                                  num_cores=1, num_subcores=NS)
    f = pl.kernel(
        _sc_body,
        out_type=jax.ShapeDtypeStruct((NS * L,), jnp.float32),
        mesh=mesh,
        compiler_params=pltpu.CompilerParams(use_tc_tiling_on_sc=False,
                                             needs_layout_passes=False),
        scratch_types=[
            pltpu.VMEM((R, D), jnp.float32),
            pltpu.VMEM((R, DP), jnp.float32),
            pltpu.VMEM((NCHUNK, CHUNK), jnp.int32),
            pltpu.VMEM((CP, D), jnp.float32),
            pltpu.VMEM((CP,), jnp.float32),
            pltpu.VMEM((CHUNK,), jnp.float32),
            pltpu.VMEM((CP // NS, D), jnp.float32),
            pltpu.VMEM((CP // NS,), jnp.float32),
            pltpu.VMEM((L,), jnp.float32),
            pltpu.VMEM_SHARED((CP, D), jnp.float32),
            pltpu.VMEM_SHARED((CP,), jnp.float32),
            pltpu.SemaphoreType.DMA,
            pltpu.SemaphoreType.DMA,
            pltpu.SemaphoreType.DMA,
            pltpu.SemaphoreType.DMA,
        ],
    )
    return f(embeddeds, tgt3d)


def kernel(embeddeds, target):
    tgt3d = target.astype(jnp.int32).reshape(NS, NCHUNK, CHUNK)
    out = _center_loss_sc(embeddeds, tgt3d)
    return jnp.sum(out)


# speedup vs baseline: 1.0886x; 1.0886x over previous
import functools

import jax
import jax.numpy as jnp
from jax import lax
from jax.experimental import pallas as pl
from jax.experimental.pallas import tpu as pltpu
from jax.experimental.pallas import tpu_sc as plsc

N = 16384
D = 32
DP = 33
C = 1000
CP = 1024
NS = 16
R = N // NS
CHUNK = 128
NCHUNK = R // CHUNK
L = 16


def _lanes_f32(val):
    return jnp.full((L,), val, dtype=jnp.float32)


def _sc_body(emb_hbm, tgt_hbm, out_hbm,
             emb_v, emb33_v, tgt_v, sums_l, cnts_l, ones_v, zrow_v, zcnt_v,
             part_v, sums_sh, cnts_sh, sem_e, sem_p, sem_t, sem_s):
    wid = lax.axis_index("s")
    base = wid * R
    zero16 = _lanes_f32(0.0)

    cp_emb = pltpu.async_copy(emb_hbm.at[pl.ds(base, R), :], emb_v, sem_e)
    cp_pad = pltpu.async_copy(emb_hbm.at[pl.ds(base, R), :],
                              emb33_v.at[:, pl.ds(0, D)], sem_p)
    cp_tgt = pltpu.async_copy(tgt_hbm.at[wid], tgt_v, sem_t)

    for i in range(CP // NS):
        zrow_v[i, pl.ds(0, L)] = zero16
        zrow_v[i, pl.ds(L, L)] = zero16
    for i in range((CP // NS) // L):
        zcnt_v[pl.ds(i * L, L)] = zero16
    for i in range(CHUNK // L):
        ones_v[pl.ds(i * L, L)] = _lanes_f32(1.0)
    pltpu.sync_copy(zrow_v, sums_sh.at[pl.ds(wid * (CP // NS), CP // NS), :])
    pltpu.sync_copy(zcnt_v, cnts_sh.at[pl.ds(wid * (CP // NS), CP // NS)])
    cp_emb.wait()
    cp_tgt.wait()
    plsc.subcore_barrier()

    adds = []
    for j in range(NCHUNK):
        idx = tgt_v.at[j]
        adds.append(pltpu.async_copy(emb_v.at[pl.ds(j * CHUNK, CHUNK), :],
                                     sums_sh.at[idx], sem_s, add=True))
        adds.append(pltpu.async_copy(ones_v, cnts_sh.at[idx], sem_s,
                                     add=True))
    for a in adds:
        a.wait()
    plsc.subcore_barrier()

    pltpu.sync_copy(sums_sh, sums_l)
    pltpu.sync_copy(cnts_sh, cnts_l)
    cp_pad.wait()

    lane = lax.broadcasted_iota(jnp.int32, (L,), 0)

    DSUB = 8

    def group_body(g, acc):
        rv = g * L + lane
        tv = plsc.load_gather(tgt_v, [lax.shift_right_logical(rv, 7),
                                      rv & (CHUNK - 1)])
        cv = plsc.load_gather(cnts_l, [tv])
        invc = 1.0 / jnp.maximum(cv, 1.0)

        def dim_body(j, sqs):
            sq0, sq1 = sqs
            d0 = j * DSUB
            for k in range(DSUB):
                dv = d0 + k
                dvv = jnp.full((L,), 0, dtype=jnp.int32) + dv
                ev = plsc.load_gather(emb33_v, [rv, dvv])
                sv = plsc.load_gather(sums_l, [tv, dvv])
                diff = ev - invc * sv
                if k & 1:
                    sq1 = sq1 + diff * diff
                else:
                    sq0 = sq0 + diff * diff
            return (sq0, sq1)

        sq0, sq1 = lax.fori_loop(0, D // DSUB, dim_body,
                                 (_lanes_f32(0.0), _lanes_f32(0.0)))
        s = sq0 + sq1
        i = plsc.bitcast(s, jnp.int32)
        i = 0x5F3759DF - lax.shift_right_logical(i, 1)
        y = plsc.bitcast(i, jnp.float32)
        for _ in range(3):
            y = y * (1.5 - 0.5 * s * y * y)
        norm = jnp.where(s > 0.0, s * y, 0.0)
        return acc + norm * invc

    part = lax.fori_loop(0, R // L, group_body, _lanes_f32(0.0))
    part_v[...] = part
    pltpu.sync_copy(part_v, out_hbm.at[pl.ds(wid * L, L)])


@jax.jit
def _center_loss_sc(embeddeds, tgt3d):
    mesh = plsc.VectorSubcoreMesh(core_axis_name="c", subcore_axis_name="s",
                                  num_cores=1, num_subcores=NS)
    f = pl.kernel(
        _sc_body,
        out_type=jax.ShapeDtypeStruct((NS * L,), jnp.float32),
        mesh=mesh,
        compiler_params=pltpu.CompilerParams(use_tc_tiling_on_sc=False,
                                             needs_layout_passes=False),
        scratch_types=[
            pltpu.VMEM((R, D), jnp.float32),
            pltpu.VMEM((R, DP), jnp.float32),
            pltpu.VMEM((NCHUNK, CHUNK), jnp.int32),
            pltpu.VMEM((CP, D), jnp.float32),
            pltpu.VMEM((CP,), jnp.float32),
            pltpu.VMEM((CHUNK,), jnp.float32),
            pltpu.VMEM((CP // NS, D), jnp.float32),
            pltpu.VMEM((CP // NS,), jnp.float32),
            pltpu.VMEM((L,), jnp.float32),
            pltpu.VMEM_SHARED((CP, D), jnp.float32),
            pltpu.VMEM_SHARED((CP,), jnp.float32),
            pltpu.SemaphoreType.DMA,
            pltpu.SemaphoreType.DMA,
            pltpu.SemaphoreType.DMA,
            pltpu.SemaphoreType.DMA,
        ],
    )
    return f(embeddeds, tgt3d)


def kernel(embeddeds, target):
    tgt3d = target.astype(jnp.int32).reshape(NS, NCHUNK, CHUNK)
    out = _center_loss_sc(embeddeds, tgt3d)
    return jnp.sum(out)
